# baseline (speedup 1.0000x reference)
import functools

import jax
import jax.numpy as jnp
from jax import lax
from jax.experimental import pallas as pl
from jax.experimental.pallas import tpu as pltpu
from jax.experimental.pallas import tpu_sc as plsc

E = 320000
V = 10000
D = 128
L = 16
NC = 2
NS = 16
NW = NC * NS
EPW = E // NW
C = 80
IB = 2000
NB = EPW // IB
NCB = IB // C
DL = D // L

_mesh = plsc.VectorSubcoreMesh(core_axis_name="c", subcore_axis_name="s")


@functools.partial(
    pl.kernel,
    mesh=_mesh,
    out_type=[
        jax.ShapeDtypeStruct((NW, L), jnp.float32),
        jax.ShapeDtypeStruct((NW, L), jnp.float32),
    ],
    scratch_types=[
        pltpu.VMEM_SHARED((V, D), jnp.float32),
        pltpu.VMEM((IB,), jnp.int32),
        pltpu.VMEM((IB,), jnp.int32),
        pltpu.VMEM((C, D), jnp.float32),
        pltpu.VMEM((C, D), jnp.float32),
        pltpu.VMEM((C, D), jnp.float32),
        pltpu.VMEM((C, D), jnp.float32),
        pltpu.VMEM((L,), jnp.float32),
        pltpu.VMEM((L,), jnp.float32),
        pltpu.SemaphoreType.DMA,
        pltpu.SemaphoreType.DMA,
    ],
)
def _edge_partials(pred_hbm, src_hbm, dst_hbm, sum_out, cnt_out,
                   pred_sp, sidx, didx, srowsA, drowsA, srowsB, drowsB,
                   sum_v, cnt_v, semA, semB):
    sid = lax.axis_index("s")
    wid = sid * NC + lax.axis_index("c")
    base0 = wid * EPW
    zeros = jnp.zeros((L,), jnp.float32)

    vps = 632

    @pl.when(sid < NS - 1)
    def _copy_main():
        pltpu.sync_copy(pred_hbm.at[pl.ds(sid * vps, vps)],
                        pred_sp.at[pl.ds(sid * vps, vps)])

    @pl.when(sid == NS - 1)
    def _copy_tail():
        pltpu.sync_copy(pred_hbm.at[pl.ds((NS - 1) * vps, V - (NS - 1) * vps)],
                        pred_sp.at[pl.ds((NS - 1) * vps, V - (NS - 1) * vps)])

    plsc.subcore_barrier()

    def issue(chunk, srows, drows, sem):
        pltpu.async_copy(pred_sp.at[sidx.at[pl.ds(chunk * C, C)]], srows, sem)
        pltpu.async_copy(pred_sp.at[didx.at[pl.ds(chunk * C, C)]], drows, sem)

    def drain(srows, drows, sem):
        pltpu.make_async_copy(pred_sp.at[pl.ds(0, C)], srows, sem).wait()
        pltpu.make_async_copy(pred_sp.at[pl.ds(0, C)], drows, sem).wait()

    def accum(srows, drows, accs):
        def edge_body(e, accs):
            new = []
            for j in range(DL):
                a = srows[e, pl.ds(j * L, L)]
                b = drows[e, pl.ds(j * L, L)]
                diff = a - b
                new.append(accs[j] + diff * diff)
            return tuple(new)
        return tuple(lax.fori_loop(0, C, edge_body, accs))

    accs = tuple(zeros for _ in range(DL))
    cnt = zeros

    for b in range(NB):
        base = base0 + b * IB
        pltpu.sync_copy(src_hbm.at[pl.ds(base, IB)], sidx)
        pltpu.sync_copy(dst_hbm.at[pl.ds(base, IB)], didx)

        issue(0, srowsA, drowsA, semA)

        def cnt_body(k, c):
            s = sidx[pl.ds(k * L, L)]
            d = didx[pl.ds(k * L, L)]
            m = (s != 0) | (d != 0)
            return c + jnp.where(m, 1.0, 0.0)

        cnt = lax.fori_loop(0, IB // L, cnt_body, cnt)

        def pair_body(g, accs):
            issue(2 * g + 1, srowsB, drowsB, semB)
            drain(srowsA, drowsA, semA)
            accs = accum(srowsA, drowsA, accs)
            issue(2 * g + 2, srowsA, drowsA, semA)
            drain(srowsB, drowsB, semB)
            return accum(srowsB, drowsB, accs)

        accs = lax.fori_loop(0, NCB // 2, pair_body, accs)
        drain(srowsA, drowsA, semA)
        accs = accum(srowsA, drowsA, accs)

    tot = accs[0]
    for j in range(1, DL):
        tot = tot + accs[j]
    sum_v[...] = tot
    cnt_v[...] = cnt
    pltpu.sync_copy(sum_v, sum_out.at[wid])
    pltpu.sync_copy(cnt_v, cnt_out.at[wid])


def _finalize_body(sums_ref, cnts_ref, out_ref):
    out_ref[0, 0] = jnp.sum(sums_ref[...]) / jnp.sum(cnts_ref[...])


_finalize = pl.pallas_call(
    _finalize_body,
    out_shape=jax.ShapeDtypeStruct((1, 1), jnp.float32),
    in_specs=[
        pl.BlockSpec(memory_space=pltpu.VMEM),
        pl.BlockSpec(memory_space=pltpu.VMEM),
    ],
    out_specs=pl.BlockSpec(memory_space=pltpu.SMEM),
)


def kernel(pred, edge_list):
    src = edge_list[0]
    dst = edge_list[1]
    sums, cnts = _edge_partials(pred, src, dst)
    return _finalize(sums, cnts)[0, 0]


# speedup vs baseline: 1.0244x; 1.0244x over previous
import functools

import jax
import jax.numpy as jnp
from jax import lax
from jax.experimental import pallas as pl
from jax.experimental.pallas import tpu as pltpu
from jax.experimental.pallas import tpu_sc as plsc

E = 320000
V = 10000
D = 128
L = 16
NC = 2
NS = 16
NW = NC * NS
EPW = E // NW
C = 40
IB = EPW
NB = EPW // IB
NCB = IB // C
DL = D // L

_mesh = plsc.VectorSubcoreMesh(core_axis_name="c", subcore_axis_name="s")


@functools.partial(
    pl.kernel,
    mesh=_mesh,
    out_type=[
        jax.ShapeDtypeStruct((NW, L), jnp.float32),
        jax.ShapeDtypeStruct((NW, L), jnp.float32),
    ],
    scratch_types=[
        pltpu.VMEM_SHARED((V, D), jnp.float32),
        pltpu.VMEM((IB,), jnp.int32),
        pltpu.VMEM((IB,), jnp.int32),
        pltpu.VMEM((C, D), jnp.float32),
        pltpu.VMEM((C, D), jnp.float32),
        pltpu.VMEM((C, D), jnp.float32),
        pltpu.VMEM((C, D), jnp.float32),
        pltpu.VMEM((L,), jnp.float32),
        pltpu.VMEM((L,), jnp.float32),
        pltpu.SemaphoreType.DMA,
        pltpu.SemaphoreType.DMA,
    ],
)
def _edge_partials(pred_hbm, src_hbm, dst_hbm, sum_out, cnt_out,
                   pred_sp, sidx, didx, srowsA, drowsA, srowsB, drowsB,
                   sum_v, cnt_v, semA, semB):
    sid = lax.axis_index("s")
    wid = sid * NC + lax.axis_index("c")
    base0 = wid * EPW
    zeros = jnp.zeros((L,), jnp.float32)

    vps = 632

    @pl.when(sid < NS - 1)
    def _copy_main():
        pltpu.sync_copy(pred_hbm.at[pl.ds(sid * vps, vps)],
                        pred_sp.at[pl.ds(sid * vps, vps)])

    @pl.when(sid == NS - 1)
    def _copy_tail():
        pltpu.sync_copy(pred_hbm.at[pl.ds((NS - 1) * vps, V - (NS - 1) * vps)],
                        pred_sp.at[pl.ds((NS - 1) * vps, V - (NS - 1) * vps)])

    plsc.subcore_barrier()

    def issue(chunk, srows, drows, sem):
        pltpu.async_copy(pred_sp.at[sidx.at[pl.ds(chunk * C, C)]], srows, sem)
        pltpu.async_copy(pred_sp.at[didx.at[pl.ds(chunk * C, C)]], drows, sem)

    def drain(srows, drows, sem):
        pltpu.make_async_copy(pred_sp.at[pl.ds(0, C)], srows, sem).wait()
        pltpu.make_async_copy(pred_sp.at[pl.ds(0, C)], drows, sem).wait()

    def accum(srows, drows, accs):
        def edge_body(e, accs):
            new = []
            for j in range(DL):
                a = srows[e, pl.ds(j * L, L)]
                b = drows[e, pl.ds(j * L, L)]
                diff = a - b
                new.append(accs[j] + diff * diff)
            return tuple(new)
        return tuple(lax.fori_loop(0, C, edge_body, accs))

    accs = tuple(zeros for _ in range(DL))
    cnt = zeros

    for b in range(NB):
        base = base0 + b * IB
        pltpu.sync_copy(src_hbm.at[pl.ds(base, IB)], sidx)
        pltpu.sync_copy(dst_hbm.at[pl.ds(base, IB)], didx)

        issue(0, srowsA, drowsA, semA)

        def cnt_body(k, c):
            s = sidx[pl.ds(k * L, L)]
            d = didx[pl.ds(k * L, L)]
            m = (s != 0) | (d != 0)
            return c + jnp.where(m, 1.0, 0.0)

        cnt = lax.fori_loop(0, IB // L, cnt_body, cnt)

        def pair_body(g, accs):
            issue(2 * g + 1, srowsB, drowsB, semB)
            drain(srowsA, drowsA, semA)
            accs = accum(srowsA, drowsA, accs)

            @pl.when(2 * g + 2 < NCB)
            def _issue_next():
                issue(2 * g + 2, srowsA, drowsA, semA)

            drain(srowsB, drowsB, semB)
            return accum(srowsB, drowsB, accs)

        accs = lax.fori_loop(0, NCB // 2, pair_body, accs)

    tot = accs[0]
    for j in range(1, DL):
        tot = tot + accs[j]
    sum_v[...] = tot
    cnt_v[...] = cnt
    pltpu.sync_copy(sum_v, sum_out.at[wid])
    pltpu.sync_copy(cnt_v, cnt_out.at[wid])


def _finalize_body(sums_ref, cnts_ref, out_ref):
    out_ref[0, 0] = jnp.sum(sums_ref[...]) / jnp.sum(cnts_ref[...])


_finalize = pl.pallas_call(
    _finalize_body,
    out_shape=jax.ShapeDtypeStruct((1, 1), jnp.float32),
    in_specs=[
        pl.BlockSpec(memory_space=pltpu.VMEM),
        pl.BlockSpec(memory_space=pltpu.VMEM),
    ],
    out_specs=pl.BlockSpec(memory_space=pltpu.SMEM),
)


def kernel(pred, edge_list):
    src = edge_list[0]
    dst = edge_list[1]
    sums, cnts = _edge_partials(pred, src, dst)
    return _finalize(sums, cnts)[0, 0]
